# baseline (speedup 1.0000x reference)
import functools

import jax
import jax.numpy as jnp
from jax.experimental import pallas as pl
from jax.experimental.pallas import tpu as pltpu


def _arm_native_kernel(x_ref, w_ref, a_ref, c_ref, out_ref,
                       xcache, pool_vec, gate, *, inv_hw, N, KC, CS):
    n = pl.program_id(0)
    k = pl.program_id(1)

    @pl.when(n > 0)
    def _():
        prev = jax.lax.rem(n + 1, 2)
        g = gate[pl.ds(k * CS, CS), :].reshape(CS, 1, 1)
        out_ref[0] = (xcache[prev, pl.ds(k * CS, CS)] * g).astype(out_ref.dtype)

    @pl.when(n < N)
    def _():
        xb = x_ref[0]
        cur = jax.lax.rem(n, 2)
        xcache[cur, pl.ds(k * CS, CS)] = xb
        pool_vec[pl.ds(k * CS, CS), :] = (
            jnp.sum(xb, axis=(1, 2)).reshape(CS, 1))

        @pl.when(k == KC - 1)
        def _():
            pooled = pool_vec[...] * inv_hw
            conv = jnp.dot(w_ref[...], pooled,
                           preferred_element_type=jnp.float32)
            gate[...] = jax.nn.sigmoid(a_ref[...] * conv + c_ref[...])


def kernel(x, conv1_w, conv1_b, bn_gamma, bn_beta, bn_mean, bn_var, eps=1e-5):
    N, C, H, W = x.shape
    CS = 128 if C % 128 == 0 else C
    KC = C // CS

    s = bn_gamma * jax.lax.rsqrt(bn_var + eps)
    a_vec = s.reshape(C, 1).astype(jnp.float32)
    c_vec = (s * (conv1_b - bn_mean) + bn_beta).reshape(C, 1).astype(jnp.float32)
    w2 = conv1_w.reshape(C, C).astype(jnp.float32)

    def xmap(n, k):
        last = n == N
        return (jnp.where(last, N - 1, n), jnp.where(last, KC - 1, k), 0, 0)

    def omap(n, k):
        first = n == 0
        return (jnp.where(first, 0, n - 1), jnp.where(first, 0, k), 0, 0)

    body = functools.partial(_arm_native_kernel, inv_hw=1.0 / (H * W),
                             N=N, KC=KC, CS=CS)
    cost = pl.CostEstimate(
        flops=int(N * (2 * C * C + 2 * C * H * W)),
        transcendentals=int(N * C),
        bytes_accessed=int(2 * N * C * H * W * 4 + C * C * 4),
    )
    out = pl.pallas_call(
        body,
        out_shape=jax.ShapeDtypeStruct((N, C, H, W), jnp.float32),
        grid=(N + 1, KC),
        in_specs=[
            pl.BlockSpec((1, CS, H, W), xmap),
            pl.BlockSpec((C, C), lambda n, k: (0, 0)),
            pl.BlockSpec((C, 1), lambda n, k: (0, 0)),
            pl.BlockSpec((C, 1), lambda n, k: (0, 0)),
        ],
        out_specs=pl.BlockSpec((1, CS, H, W), omap),
        scratch_shapes=[
            pltpu.VMEM((2, C, H, W), jnp.float32),
            pltpu.VMEM((C, 1), jnp.float32),
            pltpu.VMEM((C, 1), jnp.float32),
        ],
        compiler_params=pltpu.CompilerParams(
            dimension_semantics=("arbitrary", "arbitrary"),
            vmem_limit_bytes=58 << 20,
        ),
        cost_estimate=cost,
    )(x, w2, a_vec, c_vec)
    return out


# speedup vs baseline: 1.5564x; 1.5564x over previous
import functools

import jax
import jax.numpy as jnp
from jax.experimental import pallas as pl
from jax.experimental.pallas import tpu as pltpu


def _arm_burst_kernel(x_ref, w_ref, a_ref, c_ref, out_ref, cache, pool_acc,
                      gate, *, inv_hw, T, tile):
    p = pl.program_id(1)
    t = pl.program_id(2)

    @pl.when(p == 0)
    def _():
        xb = x_ref[0]
        cache[:, pl.ds(t * tile, tile)] = xb

        @pl.when(t == 0)
        def _():
            pool_acc[...] = jnp.zeros_like(pool_acc)

        pool_acc[...] += jnp.sum(xb, axis=1, keepdims=True)

        @pl.when(t == T - 1)
        def _():
            pooled = pool_acc[...] * inv_hw
            conv = jnp.dot(w_ref[...], pooled,
                           preferred_element_type=jnp.float32)
            gate[...] = jax.nn.sigmoid(a_ref[...] * conv + c_ref[...])

    @pl.when(p == 1)
    def _():
        out_ref[0] = (cache[:, pl.ds(t * tile, tile)]
                      * gate[...]).astype(out_ref.dtype)


def kernel(x, conv1_w, conv1_b, bn_gamma, bn_beta, bn_mean, bn_var, eps=1e-5):
    N, C, H, W = x.shape
    HW = H * W
    tile = 1024 if HW % 1024 == 0 else HW
    T = HW // tile
    x2 = x.reshape(N, C, HW)

    s = bn_gamma * jax.lax.rsqrt(bn_var + eps)
    a_vec = s.reshape(C, 1).astype(jnp.float32)
    c_vec = (s * (conv1_b - bn_mean) + bn_beta).reshape(C, 1).astype(jnp.float32)
    w2 = conv1_w.reshape(C, C).astype(jnp.float32)

    def xmap(n, p, t):
        rd = p == 0
        return (n, 0, jnp.where(rd, t, T - 1))

    def omap(n, p, t):
        rd = p == 0
        return (n, 0, jnp.where(rd, 0, t))

    body = functools.partial(_arm_burst_kernel, inv_hw=1.0 / HW, T=T,
                             tile=tile)
    cost = pl.CostEstimate(
        flops=int(N * (2 * C * C + 2 * C * HW)),
        transcendentals=int(N * C),
        bytes_accessed=int(2 * N * C * HW * 4 + C * C * 4),
    )
    out2 = pl.pallas_call(
        body,
        out_shape=jax.ShapeDtypeStruct((N, C, HW), jnp.float32),
        grid=(N, 2, T),
        in_specs=[
            pl.BlockSpec((1, C, tile), xmap),
            pl.BlockSpec((C, C), lambda n, p, t: (0, 0)),
            pl.BlockSpec((C, 1), lambda n, p, t: (0, 0)),
            pl.BlockSpec((C, 1), lambda n, p, t: (0, 0)),
        ],
        out_specs=pl.BlockSpec((1, C, tile), omap),
        scratch_shapes=[
            pltpu.VMEM((C, HW), jnp.float32),
            pltpu.VMEM((C, 1), jnp.float32),
            pltpu.VMEM((C, 1), jnp.float32),
        ],
        compiler_params=pltpu.CompilerParams(
            dimension_semantics=("parallel", "arbitrary", "arbitrary"),
            vmem_limit_bytes=52 << 20,
        ),
        cost_estimate=cost,
    )(x2, w2, a_vec, c_vec)
    return out2.reshape(N, C, H, W)


# speedup vs baseline: 1.7911x; 1.1508x over previous
import functools

import jax
import jax.numpy as jnp
from jax.experimental import pallas as pl
from jax.experimental.pallas import tpu as pltpu


def _arm_kernel(x_ref, w_ref, b_ref, gamma_ref, beta_ref, mean_ref, var_ref,
                out_ref, *, inv_hw, eps):
    xb = x_ref[0]
    pooled = jnp.sum(xb, axis=1, keepdims=True) * inv_hw
    conv = jnp.dot(w_ref[...], pooled,
                   preferred_element_type=jnp.float32) + b_ref[...]
    s = gamma_ref[...] * jax.lax.rsqrt(var_ref[...] + eps)
    z = s * (conv - mean_ref[...]) + beta_ref[...]
    gate = jax.nn.sigmoid(z)
    out_ref[0] = (xb * gate).astype(out_ref.dtype)


def kernel(x, conv1_w, conv1_b, bn_gamma, bn_beta, bn_mean, bn_var, eps=1e-5):
    N, C, H, W = x.shape
    HW = H * W
    x2 = x.reshape(N, C, HW)
    w2 = conv1_w.reshape(C, C)
    col = lambda v: v.reshape(C, 1)

    body = functools.partial(_arm_kernel, inv_hw=1.0 / HW, eps=eps)
    cost = pl.CostEstimate(
        flops=int(N * (2 * C * C + 2 * C * HW)),
        transcendentals=int(N * C),
        bytes_accessed=int(2 * N * C * HW * 4 + C * C * 4),
    )
    vec_spec = pl.BlockSpec((C, 1), lambda n: (0, 0))
    out2 = pl.pallas_call(
        body,
        out_shape=jax.ShapeDtypeStruct((N, C, HW), jnp.float32),
        grid=(N,),
        in_specs=[
            pl.BlockSpec((1, C, HW), lambda n: (n, 0, 0)),
            pl.BlockSpec((C, C), lambda n: (0, 0)),
            vec_spec, vec_spec, vec_spec, vec_spec, vec_spec,
        ],
        out_specs=pl.BlockSpec((1, C, HW), lambda n: (n, 0, 0)),
        compiler_params=pltpu.CompilerParams(
            dimension_semantics=("parallel",),
            vmem_limit_bytes=52 << 20,
        ),
        cost_estimate=cost,
    )(x2, w2, col(conv1_b), col(bn_gamma), col(bn_beta), col(bn_mean),
      col(bn_var))
    return out2.reshape(N, C, H, W)


# speedup vs baseline: 1.8175x; 1.0147x over previous
import jax
import jax.numpy as jnp
from jax.experimental import pallas as pl
from jax.experimental.pallas import tpu as pltpu


def _se_gate_kernel(x_ref, w_ref, a_ref, c_ref, out_ref, *, inv_hw):
    xb = x_ref[0]
    pooled = jnp.sum(xb, axis=1, keepdims=True) * inv_hw
    conv = jnp.dot(w_ref[...], pooled,
                   preferred_element_type=jnp.float32)
    gate = jax.nn.sigmoid(a_ref[...] * conv + c_ref[...])
    out_ref[0] = (xb * gate).astype(out_ref.dtype)


def kernel(x, conv1_w, conv1_b, bn_gamma, bn_beta, bn_mean, bn_var, eps=1e-5):
    N, C, H, W = x.shape
    HW = H * W
    x2 = x.reshape(N, C, HW)

    s = bn_gamma * jax.lax.rsqrt(bn_var + eps)
    a_vec = s.reshape(C, 1).astype(jnp.float32)
    c_vec = (s * (conv1_b - bn_mean) + bn_beta).reshape(C, 1).astype(jnp.float32)
    w2 = conv1_w.reshape(C, C).astype(jnp.float32)

    block_bytes = C * HW * 4
    vmem_bytes = 4 * block_bytes + (C * C + 2 * C) * 4 + (2 << 20)
    cost = pl.CostEstimate(
        flops=int(N * (2 * C * C + 2 * C * HW)),
        transcendentals=int(N * C),
        bytes_accessed=int(2 * N * C * HW * 4 + C * C * 4),
    )
    out2 = pl.pallas_call(
        lambda xr, wr, ar, cr, orr: _se_gate_kernel(
            xr, wr, ar, cr, orr, inv_hw=1.0 / HW),
        out_shape=jax.ShapeDtypeStruct((N, C, HW), jnp.float32),
        grid=(N,),
        in_specs=[
            pl.BlockSpec((1, C, HW), lambda n: (n, 0, 0)),
            pl.BlockSpec((C, C), lambda n: (0, 0)),
            pl.BlockSpec((C, 1), lambda n: (0, 0)),
            pl.BlockSpec((C, 1), lambda n: (0, 0)),
        ],
        out_specs=pl.BlockSpec((1, C, HW), lambda n: (n, 0, 0)),
        compiler_params=pltpu.CompilerParams(
            dimension_semantics=("parallel",),
            vmem_limit_bytes=52 << 20,
        ),
        cost_estimate=cost,
    )(x2, w2, a_vec, c_vec)

    return out2.reshape(N, C, H, W)
